# speedup vs baseline: 1.0494x; 1.0494x over previous
import functools

import jax
import jax.numpy as jnp
from jax.experimental import pallas as pl
from jax.experimental.pallas import tpu as pltpu

N_TOKENS = 32768
D_MODEL = 768
N_EXPERTS = 64
N_ACTIVE = 8
BLOCK_ROWS = 1024


def _router_block(x_ref, w_ref, probs_ref, vals_ref, idx_ref):
    x = x_ref[...]
    w = w_ref[...]
    logits = jax.lax.dot_general(
        x, w, (((1,), (1,)), ((), ())), preferred_element_type=jnp.float32
    )
    m = jnp.max(logits, axis=1, keepdims=True)
    e = jnp.exp(logits - m)
    s = jnp.sum(e, axis=1, keepdims=True)
    probs = e / s
    probs_ref[...] = probs

    expert_iota = jax.lax.broadcasted_iota(jnp.int32, probs.shape, 1)
    work = probs
    vals_cols = []
    idx_cols = []
    for _ in range(N_ACTIVE):
        mx = jnp.max(work, axis=1, keepdims=True)
        amx = jnp.min(
            jnp.where(work == mx, expert_iota, N_EXPERTS), axis=1, keepdims=True
        )
        vals_cols.append(mx)
        idx_cols.append(amx)
        work = jnp.where(expert_iota == amx, -jnp.inf, work)
    vals = jnp.concatenate(vals_cols, axis=1)
    idx = jnp.concatenate(idx_cols, axis=1)
    vals = vals / (jnp.sum(vals, axis=1, keepdims=True) + 1e-6)
    vals_ref[...] = vals
    idx_ref[...] = idx


@jax.jit
def kernel(x, W):
    n = x.shape[0]
    grid = (n // BLOCK_ROWS,)
    probs, vals, idx = pl.pallas_call(
        _router_block,
        grid=grid,
        in_specs=[
            pl.BlockSpec((BLOCK_ROWS, D_MODEL), lambda i: (i, 0)),
            pl.BlockSpec((N_EXPERTS, D_MODEL), lambda i: (0, 0)),
        ],
        out_specs=[
            pl.BlockSpec((BLOCK_ROWS, N_EXPERTS), lambda i: (i, 0)),
            pl.BlockSpec((BLOCK_ROWS, N_ACTIVE), lambda i: (i, 0)),
            pl.BlockSpec((BLOCK_ROWS, N_ACTIVE), lambda i: (i, 0)),
        ],
        out_shape=[
            jax.ShapeDtypeStruct((n, N_EXPERTS), jnp.float32),
            jax.ShapeDtypeStruct((n, N_ACTIVE), jnp.float32),
            jax.ShapeDtypeStruct((n, N_ACTIVE), jnp.int32),
        ],
    )(x, W)
    return (vals, idx, probs)


# speedup vs baseline: 1.8828x; 1.7942x over previous
import functools

import jax
import jax.numpy as jnp
from jax.experimental import pallas as pl
from jax.experimental.pallas import tpu as pltpu

N_TOKENS = 32768
D_MODEL = 768
N_EXPERTS = 64
N_ACTIVE = 8
BLOCK_ROWS = 1024


def _router_block(x_ref, w_ref, probs_ref, vals_ref, idx_ref):
    x = x_ref[...]
    w = w_ref[...]
    lt = jax.lax.dot_general(
        w, x, (((1,), (1,)), ((), ())), preferred_element_type=jnp.float32
    )
    m = jnp.max(lt, axis=0, keepdims=True)
    et = jnp.exp(lt - m)
    s = jnp.sum(et, axis=0, keepdims=True)
    pt = et / s
    probs_ref[...] = pt.T

    eio = jax.lax.broadcasted_iota(jnp.int32, pt.shape, 0)
    work = pt
    vals_rows = []
    idx_rows = []
    for _ in range(N_ACTIVE):
        mx = jnp.max(work, axis=0, keepdims=True)
        amx = jnp.min(
            jnp.where(work == mx, eio, N_EXPERTS), axis=0, keepdims=True
        )
        vals_rows.append(mx)
        idx_rows.append(amx)
        work = jnp.where(eio == amx, -jnp.inf, work)
    vt = jnp.concatenate(vals_rows, axis=0)
    it = jnp.concatenate(idx_rows, axis=0)
    vt = vt / (jnp.sum(vt, axis=0, keepdims=True) + 1e-6)
    vals_ref[...] = vt.T
    idx_ref[...] = it.T


@jax.jit
def kernel(x, W):
    n = x.shape[0]
    grid = (n // BLOCK_ROWS,)
    probs, vals, idx = pl.pallas_call(
        _router_block,
        grid=grid,
        in_specs=[
            pl.BlockSpec((BLOCK_ROWS, D_MODEL), lambda i: (i, 0)),
            pl.BlockSpec((N_EXPERTS, D_MODEL), lambda i: (0, 0)),
        ],
        out_specs=[
            pl.BlockSpec((BLOCK_ROWS, N_EXPERTS), lambda i: (i, 0)),
            pl.BlockSpec((BLOCK_ROWS, N_ACTIVE), lambda i: (i, 0)),
            pl.BlockSpec((BLOCK_ROWS, N_ACTIVE), lambda i: (i, 0)),
        ],
        out_shape=[
            jax.ShapeDtypeStruct((n, N_EXPERTS), jnp.float32),
            jax.ShapeDtypeStruct((n, N_ACTIVE), jnp.float32),
            jax.ShapeDtypeStruct((n, N_ACTIVE), jnp.int32),
        ],
    )(x, W)
    return (vals, idx, probs)


# speedup vs baseline: 2.0407x; 1.0839x over previous
import functools

import jax
import jax.numpy as jnp
from jax.experimental import pallas as pl
from jax.experimental.pallas import tpu as pltpu

N_TOKENS = 32768
D_MODEL = 768
N_EXPERTS = 64
N_ACTIVE = 8
BLOCK_ROWS = 2048


def _router_block(x_ref, w_ref, probs_ref, vals_ref, idx_ref):
    x = x_ref[...]
    w = w_ref[...]
    lt = jax.lax.dot_general(
        w, x, (((1,), (1,)), ((), ())), preferred_element_type=jnp.float32
    )
    m = jnp.max(lt, axis=0, keepdims=True)
    et = jnp.exp(lt - m)
    s = jnp.sum(et, axis=0, keepdims=True)
    pt = et / s
    probs_ref[...] = pt.T

    eio = jax.lax.broadcasted_iota(jnp.int32, pt.shape, 0)
    work = pt
    vals_rows = []
    idx_rows = []
    for _ in range(N_ACTIVE):
        mx = jnp.max(work, axis=0, keepdims=True)
        amx = jnp.min(
            jnp.where(work == mx, eio, N_EXPERTS), axis=0, keepdims=True
        )
        vals_rows.append(mx)
        idx_rows.append(amx)
        work = jnp.where(eio == amx, -jnp.inf, work)
    vt = jnp.concatenate(vals_rows, axis=0)
    it = jnp.concatenate(idx_rows, axis=0)
    vt = vt / (jnp.sum(vt, axis=0, keepdims=True) + 1e-6)
    vals_ref[...] = vt.T
    idx_ref[...] = it.T


@jax.jit
def kernel(x, W):
    n = x.shape[0]
    grid = (n // BLOCK_ROWS,)
    probs, vals, idx = pl.pallas_call(
        _router_block,
        grid=grid,
        in_specs=[
            pl.BlockSpec((BLOCK_ROWS, D_MODEL), lambda i: (i, 0)),
            pl.BlockSpec((N_EXPERTS, D_MODEL), lambda i: (0, 0)),
        ],
        out_specs=[
            pl.BlockSpec((BLOCK_ROWS, N_EXPERTS), lambda i: (i, 0)),
            pl.BlockSpec((BLOCK_ROWS, N_ACTIVE), lambda i: (i, 0)),
            pl.BlockSpec((BLOCK_ROWS, N_ACTIVE), lambda i: (i, 0)),
        ],
        out_shape=[
            jax.ShapeDtypeStruct((n, N_EXPERTS), jnp.float32),
            jax.ShapeDtypeStruct((n, N_ACTIVE), jnp.float32),
            jax.ShapeDtypeStruct((n, N_ACTIVE), jnp.int32),
        ],
    )(x, W)
    return (vals, idx, probs)


# speedup vs baseline: 2.1366x; 1.0470x over previous
import functools

import jax
import jax.numpy as jnp
from jax.experimental import pallas as pl
from jax.experimental.pallas import tpu as pltpu

N_TOKENS = 32768
D_MODEL = 768
N_EXPERTS = 64
N_ACTIVE = 8
BLOCK_ROWS = 4096


def _router_block(x_ref, w_ref, probs_ref, vals_ref, idx_ref):
    x = x_ref[...]
    w = w_ref[...]
    lt = jax.lax.dot_general(
        w, x, (((1,), (1,)), ((), ())), preferred_element_type=jnp.float32
    )
    m = jnp.max(lt, axis=0, keepdims=True)
    et = jnp.exp(lt - m)
    s = jnp.sum(et, axis=0, keepdims=True)
    pt = et / s
    probs_ref[...] = pt.T

    eio = jax.lax.broadcasted_iota(jnp.int32, pt.shape, 0)
    work = pt
    vals_rows = []
    idx_rows = []
    for _ in range(N_ACTIVE):
        mx = jnp.max(work, axis=0, keepdims=True)
        amx = jnp.min(
            jnp.where(work == mx, eio, N_EXPERTS), axis=0, keepdims=True
        )
        vals_rows.append(mx)
        idx_rows.append(amx)
        work = jnp.where(eio == amx, -jnp.inf, work)
    vt = jnp.concatenate(vals_rows, axis=0)
    it = jnp.concatenate(idx_rows, axis=0)
    vt = vt / (jnp.sum(vt, axis=0, keepdims=True) + 1e-6)
    vals_ref[...] = vt.T
    idx_ref[...] = it.T


@jax.jit
def kernel(x, W):
    n = x.shape[0]
    grid = (n // BLOCK_ROWS,)
    probs, vals, idx = pl.pallas_call(
        _router_block,
        grid=grid,
        in_specs=[
            pl.BlockSpec((BLOCK_ROWS, D_MODEL), lambda i: (i, 0)),
            pl.BlockSpec((N_EXPERTS, D_MODEL), lambda i: (0, 0)),
        ],
        out_specs=[
            pl.BlockSpec((BLOCK_ROWS, N_EXPERTS), lambda i: (i, 0)),
            pl.BlockSpec((BLOCK_ROWS, N_ACTIVE), lambda i: (i, 0)),
            pl.BlockSpec((BLOCK_ROWS, N_ACTIVE), lambda i: (i, 0)),
        ],
        out_shape=[
            jax.ShapeDtypeStruct((n, N_EXPERTS), jnp.float32),
            jax.ShapeDtypeStruct((n, N_ACTIVE), jnp.float32),
            jax.ShapeDtypeStruct((n, N_ACTIVE), jnp.int32),
        ],
    )(x, W)
    return (vals, idx, probs)
